# baseline (speedup 1.0000x reference)
import functools

import jax
import jax.numpy as jnp
from jax import lax
from jax.experimental import pallas as pl
from jax.experimental.pallas import tpu as pltpu
from jax.experimental.pallas import tpu_sc as plsc

N = 10000
E = 320000
D = 128
HL = 64
B = 256
NL = 3
BN_EPS = 1e-5

NC = 2
NS = 16
NW = NC * NS
CH = 125
ROWS = E // (NW * CH)
NP = 10240
NPT = NP // NS
ZROWS = 64

_HIGH = jax.lax.Precision.HIGHEST


NB = 8
NG = 4


def _zero_acc(zbuf, acc, sid):
    f = zbuf.shape[1]
    zeros16 = jnp.zeros((16,), jnp.float32)

    def _zb(k, carry):
        r = k // (f // 16)
        col = (k % (f // 16)) * 16
        zbuf[r, pl.ds(col, 16)] = zeros16
        return carry

    lax.fori_loop(0, ZROWS * (f // 16), _zb, 0)
    for j in range(NPT // ZROWS):
        pltpu.sync_copy(zbuf, acc.at[pl.ds(sid * NPT + j * ZROWS, ZROWS)])


def _edge_pipeline(x_hbm, srcv, dstv, rows, acc, gsems, ssems, nch):
    def _gather(c, b):
        pltpu.async_copy(x_hbm.at[srcv.at[c]], rows[b], gsems[b])

    def _gwait(c, b):
        pltpu.make_async_copy(x_hbm.at[srcv.at[c]], rows[b], gsems[b]).wait()

    def _scat(c, b):
        pltpu.async_copy(rows[b], acc.at[dstv.at[c]], ssems[b], add=True)

    def _swait(c, b):
        pltpu.make_async_copy(rows[b], acc.at[dstv.at[c]], ssems[b]).wait()

    for c in range(NG):
        _gather(c, c)
    for b in range(NB):
        _gwait(b, b)
        _scat(b, b)
        if b >= NB - NG:
            _swait(b - (NB - NG), (b + NG) % NB)
        _gather(b + NG, (b + NG) % NB)

    def _group(g, carry):
        c0 = NB * g
        for b in range(NB):
            c = c0 + b
            _gwait(c, b)
            _scat(c, b)
            _swait(c - (NB - NG), (b + NG) % NB)
            _gather(c + NG, (b + NG) % NB)
        return carry

    lax.fori_loop(1, nch // NB - 1, _group, 0)
    c0 = nch - NB
    for b in range(NB):
        c = c0 + b
        _gwait(c, b)
        _scat(c, b)
        _swait(c - (NB - NG), (b + NG) % NB)
        if c + NG < nch:
            _gather(c + NG, (b + NG) % NB)
    for c in range(nch - (NB - NG), nch):
        _swait(c, c % NB)


def _sc_body(x_hbm, ei_hbm, out_hbm, srcv, dstv, rows, zbuf, acc,
             gsems, ssems):
    cid = lax.axis_index("c")
    sid = lax.axis_index("s")
    wid = sid * NC + cid

    _zero_acc(zbuf, acc, sid)
    plsc.subcore_barrier()

    for half in range(2):
        base = wid * ROWS + half * (ROWS // 2)
        pltpu.sync_copy(ei_hbm.at[0, pl.ds(base, ROWS // 2)], srcv)
        pltpu.sync_copy(ei_hbm.at[1, pl.ds(base, ROWS // 2)], dstv)
        _edge_pipeline(x_hbm, srcv, dstv, rows, acc, gsems, ssems, ROWS // 2)

    plsc.subcore_barrier()
    pltpu.sync_copy(acc.at[pl.ds(sid * NPT, NPT)],
                    out_hbm.at[cid, pl.ds(sid * NPT, NPT)])


ROWS0 = (E // CH) // NS


def _sc_body0(xlo_hbm, xhi_hbm, ei_hbm, out_hbm, srcv, dstv, rows, zbuf, acc,
              gsems, ssems):
    cid = lax.axis_index("c")
    sid = lax.axis_index("s")

    _zero_acc(zbuf, acc, sid)
    plsc.subcore_barrier()

    for half in range(2):
        base = sid * ROWS0 + half * (ROWS0 // 2)
        pltpu.sync_copy(ei_hbm.at[0, pl.ds(base, ROWS0 // 2)], srcv)
        pltpu.sync_copy(ei_hbm.at[1, pl.ds(base, ROWS0 // 2)], dstv)

        @pl.when(cid == 0)
        def _():
            _edge_pipeline(xlo_hbm, srcv, dstv, rows, acc, gsems, ssems,
                           ROWS0 // 2)

        @pl.when(cid == 1)
        def _():
            _edge_pipeline(xhi_hbm, srcv, dstv, rows, acc, gsems, ssems,
                           ROWS0 // 2)

    plsc.subcore_barrier()
    pltpu.sync_copy(acc.at[pl.ds(sid * NPT, NPT)],
                    out_hbm.at[cid, pl.ds(sid * NPT, NPT)])


def _sc_mesh():
    return plsc.VectorSubcoreMesh(core_axis_name="c", subcore_axis_name="s",
                                  num_cores=NC, num_subcores=NS)


def _sc_scratch(nrows, f):
    return [
        pltpu.VMEM((nrows, CH), jnp.int32),
        pltpu.VMEM((nrows, CH), jnp.int32),
        [pltpu.VMEM((CH, f), jnp.float32) for _ in range(NB)],
        pltpu.VMEM((ZROWS, f), jnp.float32),
        pltpu.VMEM_SHARED((NP, f), jnp.float32),
        [pltpu.SemaphoreType.DMA for _ in range(NB)],
        [pltpu.SemaphoreType.DMA for _ in range(NB)],
    ]


@functools.lru_cache(maxsize=None)
def _get_sc_segsum():
    return pl.kernel(
        _sc_body,
        out_type=jax.ShapeDtypeStruct((NC, NP, HL), jnp.float32),
        mesh=_sc_mesh(),
        compiler_params=pltpu.CompilerParams(use_tc_tiling_on_sc=False),
        scratch_types=_sc_scratch(ROWS // 2, HL),
    )


@functools.lru_cache(maxsize=None)
def _get_sc_segsum0():
    return pl.kernel(
        _sc_body0,
        out_type=jax.ShapeDtypeStruct((NC, NP, HL), jnp.float32),
        mesh=_sc_mesh(),
        compiler_params=pltpu.CompilerParams(use_tc_tiling_on_sc=False),
        scratch_types=_sc_scratch(ROWS0 // 2, HL),
    )


def _sc_segsum(x, ei3):
    return _get_sc_segsum()(x, ei3)


def _sc_segsum0(xlo, xhi, ei3):
    return _get_sc_segsum0()(xlo, xhi, ei3)


_RB = 400


def _make_layer_body(concat_agg):
  def _layer_body(eps_ref, x_ref, agg_ref, batch_ref, w1_ref, b1_ref,
                  gamma_ref, beta_ref, w2_ref, b2_ref, xnext_ref, pooled_ref):
    if concat_agg:
        agg = jnp.concatenate([agg_ref[0], agg_ref[1]], axis=1)
    else:
        agg = agg_ref[0] + agg_ref[1]
    i = pl.program_id(0)
    h = (1.0 + eps_ref[0]) * x_ref[...] + agg
    h = lax.dot(h, w1_ref[...], preferred_element_type=jnp.float32)
    h = h + b1_ref[...]
    h = (h / jnp.sqrt(1.0 + BN_EPS)) * gamma_ref[...] + beta_ref[...]
    h = jnp.maximum(h, 0.0)
    h = lax.dot(h, w2_ref[...], preferred_element_type=jnp.float32)
    x1 = jnp.maximum(h + b2_ref[...], 0.0)
    xnext_ref[...] = x1
    b = batch_ref[0, 0, :]
    onehot = (b[:, None] == lax.broadcasted_iota(jnp.int32, (1, B), 1)
              ).astype(jnp.float32)
    contrib = lax.dot_general(onehot, x1, (((0,), (0,)), ((), ())),
                              precision=_HIGH,
                              preferred_element_type=jnp.float32)

    @pl.when(i == 0)
    def _():
        pooled_ref[...] = contrib

    @pl.when(i != 0)
    def _():
        pooled_ref[...] += contrib
  return _layer_body


def _tc_layer(eps, x, aggp, batch3, w1, b1, gamma, beta, w2, b2):
    f = x.shape[1]
    return pl.pallas_call(
        _make_layer_body(f == D),
        grid=(N // _RB,),
        in_specs=[
            pl.BlockSpec(memory_space=pltpu.SMEM),
            pl.BlockSpec((_RB, f), lambda i: (i, 0)),
            pl.BlockSpec((NC, _RB, HL), lambda i: (0, i, 0)),
            pl.BlockSpec((1, 1, _RB), lambda i: (i, 0, 0)),
            pl.BlockSpec((f, HL), lambda i: (0, 0)),
            pl.BlockSpec((1, HL), lambda i: (0, 0)),
            pl.BlockSpec((1, HL), lambda i: (0, 0)),
            pl.BlockSpec((1, HL), lambda i: (0, 0)),
            pl.BlockSpec((HL, HL), lambda i: (0, 0)),
            pl.BlockSpec((1, HL), lambda i: (0, 0)),
        ],
        out_specs=[
            pl.BlockSpec((_RB, HL), lambda i: (i, 0)),
            pl.BlockSpec((B, HL), lambda i: (0, 0)),
        ],
        out_shape=[
            jax.ShapeDtypeStruct((N, HL), jnp.float32),
            jax.ShapeDtypeStruct((B, HL), jnp.float32),
        ],
    )(eps, x, aggp, batch3, w1, b1, gamma, beta, w2, b2)


def _head_body(p0_ref, p1_ref, p2_ref, idx_ref, conc_ref, ea_ref, ec_ref,
               w1_ref, b1_ref, w2_ref, b2_ref, o_ref):
    w = w1_ref[...]
    h = lax.dot(p0_ref[...], w[0:HL], preferred_element_type=jnp.float32)
    h += lax.dot(p1_ref[...], w[HL:2 * HL], preferred_element_type=jnp.float32)
    h += lax.dot(p2_ref[...], w[2 * HL:3 * HL],
                 preferred_element_type=jnp.float32)
    xc = jnp.where(idx_ref[...] == 0, ea_ref[0:1, :], ea_ref[1:2, :])
    h += lax.dot(xc, w[3 * HL:3 * HL + 128],
                 preferred_element_type=jnp.float32)
    cc = jnp.where(conc_ref[...] == 0.5, ec_ref[0:1, :], ec_ref[1:2, :])
    h += lax.dot(cc, w[3 * HL + 128:3 * HL + 256],
                 preferred_element_type=jnp.float32)
    h = jnp.maximum(h + b1_ref[...], 0.0)
    o = lax.dot(h, w2_ref[...], preferred_element_type=jnp.float32)
    o_ref[...] = jax.nn.sigmoid(o + b2_ref[...])


def _tc_head(p0, p1, p2, idx2, conc2, ea, ec, w1, b1, w2p, b2p):
    return pl.pallas_call(
        _head_body,
        out_shape=jax.ShapeDtypeStruct((B, 128), jnp.float32),
    )(p0, p1, p2, idx2, conc2, ea, ec, w1, b1, w2p, b2p)


@jax.jit
def kernel(x_a, concentration, params, edge_index, batch, indices):
    ei3 = edge_index.reshape(2, E // CH, CH)
    x = x_a
    batch3 = batch.reshape(N // _RB, 1, _RB)

    pooled = []
    for i in range(NL):
        if x.shape[1] == D:
            aggp = _sc_segsum0(x[:, 0:HL], x[:, HL:D], ei3)
        else:
            aggp = _sc_segsum(x, ei3)
        x, pool_i = _tc_layer(
            params["gin%d_eps" % i].reshape(1), x, aggp, batch3,
            params["gin%d_w1" % i], params["gin%d_b1" % i].reshape(1, HL),
            params["gin%d_gamma" % i].reshape(1, HL),
            params["gin%d_beta" % i].reshape(1, HL),
            params["gin%d_w2" % i], params["gin%d_b2" % i].reshape(1, HL))
        pooled.append(pool_i)

    w2p = jnp.zeros((HL, 128), jnp.float32).at[:, 0:2].set(params["fc2_w"])
    b2p = jnp.zeros((1, 128), jnp.float32).at[0, 0:2].set(params["fc2_b"])
    out = _tc_head(pooled[0], pooled[1], pooled[2],
                   indices.reshape(B, 1), concentration.reshape(B, 1),
                   params["emb_acid"], params["emb_conc"],
                   params["fc1_w"], params["fc1_b"].reshape(1, HL), w2p, b2p)
    return out[:, 0:2]


# speedup vs baseline: 1.0445x; 1.0445x over previous
import functools

import jax
import jax.numpy as jnp
from jax import lax
from jax.experimental import pallas as pl
from jax.experimental.pallas import tpu as pltpu
from jax.experimental.pallas import tpu_sc as plsc

N = 10000
E = 320000
D = 128
HL = 64
B = 256
NL = 3
BN_EPS = 1e-5

NC = 2
NS = 16
NW = NC * NS
CH = 125
ROWS = E // (NW * CH)
NP = 10240
NPT = NP // NS
ZROWS = 64

_HIGH = jax.lax.Precision.HIGHEST


NB = 5
NG = 3


def _zero_acc(zbuf, acc, sid):
    f = zbuf.shape[1]
    zeros16 = jnp.zeros((16,), jnp.float32)

    def _zb(k, carry):
        r = k // (f // 16)
        col = (k % (f // 16)) * 16
        zbuf[r, pl.ds(col, 16)] = zeros16
        return carry

    lax.fori_loop(0, ZROWS * (f // 16), _zb, 0)
    for j in range(NPT // ZROWS):
        pltpu.sync_copy(zbuf, acc.at[pl.ds(sid * NPT + j * ZROWS, ZROWS)])


def _edge_pipeline(x_hbm, srcv, dstv, rows, acc, gsems, ssems, nch):
    def _gather(c, b):
        pltpu.async_copy(x_hbm.at[srcv.at[c]], rows[b], gsems[b])

    def _gwait(c, b):
        pltpu.make_async_copy(x_hbm.at[srcv.at[c]], rows[b], gsems[b]).wait()

    def _scat(c, b):
        pltpu.async_copy(rows[b], acc.at[dstv.at[c]], ssems[b], add=True)

    def _swait(c, b):
        pltpu.make_async_copy(rows[b], acc.at[dstv.at[c]], ssems[b]).wait()

    for c in range(NG):
        _gather(c, c)
    for b in range(NB):
        _gwait(b, b)
        _scat(b, b)
        if b >= NB - NG:
            _swait(b - (NB - NG), (b + NG) % NB)
        _gather(b + NG, (b + NG) % NB)

    def _group(g, carry):
        c0 = NB * g
        for b in range(NB):
            c = c0 + b
            _gwait(c, b)
            _scat(c, b)
            _swait(c - (NB - NG), (b + NG) % NB)
            _gather(c + NG, (b + NG) % NB)
        return carry

    lax.fori_loop(1, nch // NB - 1, _group, 0)
    c0 = nch - NB
    for b in range(NB):
        c = c0 + b
        _gwait(c, b)
        _scat(c, b)
        _swait(c - (NB - NG), (b + NG) % NB)
        if c + NG < nch:
            _gather(c + NG, (b + NG) % NB)
    for c in range(nch - (NB - NG), nch):
        _swait(c, c % NB)


def _sc_body(x_hbm, ei_hbm, out_hbm, srcv, dstv, rows, zbuf, acc,
             gsems, ssems):
    cid = lax.axis_index("c")
    sid = lax.axis_index("s")
    wid = sid * NC + cid

    _zero_acc(zbuf, acc, sid)
    plsc.subcore_barrier()

    base = wid * ROWS
    pltpu.sync_copy(ei_hbm.at[0, pl.ds(base, ROWS)], srcv)
    pltpu.sync_copy(ei_hbm.at[1, pl.ds(base, ROWS)], dstv)
    _edge_pipeline(x_hbm, srcv, dstv, rows, acc, gsems, ssems, ROWS)

    plsc.subcore_barrier()
    pltpu.sync_copy(acc.at[pl.ds(sid * NPT, NPT)],
                    out_hbm.at[cid, pl.ds(sid * NPT, NPT)])


ROWS0 = (E // CH) // NS


def _sc_body0(xlo_hbm, xhi_hbm, ei_hbm, out_hbm, srcv, dstv, rows, zbuf, acc,
              gsems, ssems):
    cid = lax.axis_index("c")
    sid = lax.axis_index("s")

    _zero_acc(zbuf, acc, sid)
    plsc.subcore_barrier()

    for half in range(2):
        base = sid * ROWS0 + half * (ROWS0 // 2)
        pltpu.sync_copy(ei_hbm.at[0, pl.ds(base, ROWS0 // 2)], srcv)
        pltpu.sync_copy(ei_hbm.at[1, pl.ds(base, ROWS0 // 2)], dstv)

        @pl.when(cid == 0)
        def _():
            _edge_pipeline(xlo_hbm, srcv, dstv, rows, acc, gsems, ssems,
                           ROWS0 // 2)

        @pl.when(cid == 1)
        def _():
            _edge_pipeline(xhi_hbm, srcv, dstv, rows, acc, gsems, ssems,
                           ROWS0 // 2)

    plsc.subcore_barrier()
    pltpu.sync_copy(acc.at[pl.ds(sid * NPT, NPT)],
                    out_hbm.at[cid, pl.ds(sid * NPT, NPT)])


def _sc_mesh():
    return plsc.VectorSubcoreMesh(core_axis_name="c", subcore_axis_name="s",
                                  num_cores=NC, num_subcores=NS)


def _sc_scratch(nrows, f):
    return [
        pltpu.VMEM((nrows, CH), jnp.int32),
        pltpu.VMEM((nrows, CH), jnp.int32),
        [pltpu.VMEM((CH, f), jnp.float32) for _ in range(NB)],
        pltpu.VMEM((ZROWS, f), jnp.float32),
        pltpu.VMEM_SHARED((NP, f), jnp.float32),
        [pltpu.SemaphoreType.DMA for _ in range(NB)],
        [pltpu.SemaphoreType.DMA for _ in range(NB)],
    ]


@functools.lru_cache(maxsize=None)
def _get_sc_segsum():
    return pl.kernel(
        _sc_body,
        out_type=jax.ShapeDtypeStruct((NC, NP, HL), jnp.float32),
        mesh=_sc_mesh(),
        compiler_params=pltpu.CompilerParams(use_tc_tiling_on_sc=False),
        scratch_types=_sc_scratch(ROWS, HL),
    )


@functools.lru_cache(maxsize=None)
def _get_sc_segsum0():
    return pl.kernel(
        _sc_body0,
        out_type=jax.ShapeDtypeStruct((NC, NP, HL), jnp.float32),
        mesh=_sc_mesh(),
        compiler_params=pltpu.CompilerParams(use_tc_tiling_on_sc=False),
        scratch_types=_sc_scratch(ROWS0 // 2, HL),
    )


def _sc_segsum(x, ei3):
    return _get_sc_segsum()(x, ei3)


def _sc_segsum0(xlo, xhi, ei3):
    return _get_sc_segsum0()(xlo, xhi, ei3)


_RB = 400


def _make_layer_body(concat_agg):
  def _layer_body(eps_ref, x_ref, agg_ref, batch_ref, w1_ref, b1_ref,
                  gamma_ref, beta_ref, w2_ref, b2_ref, xnext_ref, pooled_ref):
    if concat_agg:
        agg = jnp.concatenate([agg_ref[0], agg_ref[1]], axis=1)
    else:
        agg = agg_ref[0] + agg_ref[1]
    i = pl.program_id(0)
    h = (1.0 + eps_ref[0]) * x_ref[...] + agg
    h = lax.dot(h, w1_ref[...], preferred_element_type=jnp.float32)
    h = h + b1_ref[...]
    h = (h / jnp.sqrt(1.0 + BN_EPS)) * gamma_ref[...] + beta_ref[...]
    h = jnp.maximum(h, 0.0)
    h = lax.dot(h, w2_ref[...], preferred_element_type=jnp.float32)
    x1 = jnp.maximum(h + b2_ref[...], 0.0)
    xnext_ref[...] = x1
    b = batch_ref[0, 0, :]
    onehot = (b[:, None] == lax.broadcasted_iota(jnp.int32, (1, B), 1)
              ).astype(jnp.float32)
    contrib = lax.dot_general(onehot, x1, (((0,), (0,)), ((), ())),
                              precision=_HIGH,
                              preferred_element_type=jnp.float32)

    @pl.when(i == 0)
    def _():
        pooled_ref[...] = contrib

    @pl.when(i != 0)
    def _():
        pooled_ref[...] += contrib
  return _layer_body


def _tc_layer(eps, x, aggp, batch3, w1, b1, gamma, beta, w2, b2):
    f = x.shape[1]
    return pl.pallas_call(
        _make_layer_body(f == D),
        grid=(N // _RB,),
        in_specs=[
            pl.BlockSpec(memory_space=pltpu.SMEM),
            pl.BlockSpec((_RB, f), lambda i: (i, 0)),
            pl.BlockSpec((NC, _RB, HL), lambda i: (0, i, 0)),
            pl.BlockSpec((1, 1, _RB), lambda i: (i, 0, 0)),
            pl.BlockSpec((f, HL), lambda i: (0, 0)),
            pl.BlockSpec((1, HL), lambda i: (0, 0)),
            pl.BlockSpec((1, HL), lambda i: (0, 0)),
            pl.BlockSpec((1, HL), lambda i: (0, 0)),
            pl.BlockSpec((HL, HL), lambda i: (0, 0)),
            pl.BlockSpec((1, HL), lambda i: (0, 0)),
        ],
        out_specs=[
            pl.BlockSpec((_RB, HL), lambda i: (i, 0)),
            pl.BlockSpec((B, HL), lambda i: (0, 0)),
        ],
        out_shape=[
            jax.ShapeDtypeStruct((N, HL), jnp.float32),
            jax.ShapeDtypeStruct((B, HL), jnp.float32),
        ],
    )(eps, x, aggp, batch3, w1, b1, gamma, beta, w2, b2)


def _head_body(p0_ref, p1_ref, p2_ref, idx_ref, conc_ref, ea_ref, ec_ref,
               w1_ref, b1_ref, w2_ref, b2_ref, o_ref):
    w = w1_ref[...]
    h = lax.dot(p0_ref[...], w[0:HL], preferred_element_type=jnp.float32)
    h += lax.dot(p1_ref[...], w[HL:2 * HL], preferred_element_type=jnp.float32)
    h += lax.dot(p2_ref[...], w[2 * HL:3 * HL],
                 preferred_element_type=jnp.float32)
    xc = jnp.where(idx_ref[...] == 0, ea_ref[0:1, :], ea_ref[1:2, :])
    h += lax.dot(xc, w[3 * HL:3 * HL + 128],
                 preferred_element_type=jnp.float32)
    cc = jnp.where(conc_ref[...] == 0.5, ec_ref[0:1, :], ec_ref[1:2, :])
    h += lax.dot(cc, w[3 * HL + 128:3 * HL + 256],
                 preferred_element_type=jnp.float32)
    h = jnp.maximum(h + b1_ref[...], 0.0)
    o = lax.dot(h, w2_ref[...], preferred_element_type=jnp.float32)
    o_ref[...] = jax.nn.sigmoid(o + b2_ref[...])


def _tc_head(p0, p1, p2, idx2, conc2, ea, ec, w1, b1, w2p, b2p):
    return pl.pallas_call(
        _head_body,
        out_shape=jax.ShapeDtypeStruct((B, 128), jnp.float32),
    )(p0, p1, p2, idx2, conc2, ea, ec, w1, b1, w2p, b2p)


@jax.jit
def kernel(x_a, concentration, params, edge_index, batch, indices):
    ei3 = edge_index.reshape(2, E // CH, CH)
    x = x_a
    batch3 = batch.reshape(N // _RB, 1, _RB)

    pooled = []
    for i in range(NL):
        if x.shape[1] == D:
            aggp = _sc_segsum0(x[:, 0:HL], x[:, HL:D], ei3)
        else:
            aggp = _sc_segsum(x, ei3)
        x, pool_i = _tc_layer(
            params["gin%d_eps" % i].reshape(1), x, aggp, batch3,
            params["gin%d_w1" % i], params["gin%d_b1" % i].reshape(1, HL),
            params["gin%d_gamma" % i].reshape(1, HL),
            params["gin%d_beta" % i].reshape(1, HL),
            params["gin%d_w2" % i], params["gin%d_b2" % i].reshape(1, HL))
        pooled.append(pool_i)

    w2p = jnp.zeros((HL, 128), jnp.float32).at[:, 0:2].set(params["fc2_w"])
    b2p = jnp.zeros((1, 128), jnp.float32).at[0, 0:2].set(params["fc2_b"])
    out = _tc_head(pooled[0], pooled[1], pooled[2],
                   indices.reshape(B, 1), concentration.reshape(B, 1),
                   params["emb_acid"], params["emb_conc"],
                   params["fc1_w"], params["fc1_b"].reshape(1, HL), w2p, b2p)
    return out[:, 0:2]
